# Initial kernel scaffold; baseline (speedup 1.0000x reference)
import jax
import jax.numpy as jnp
from jax.experimental import pallas as pl


def kernel(x, weight):
    raise NotImplementedError("write your pallas kernel here")


# speedup vs baseline: 1.0219x; 1.0219x over previous
import functools

import jax
import jax.numpy as jnp
from jax.experimental import pallas as pl
from jax.experimental.pallas import tpu as pltpu

_MiB = 1024 * 1024


def _rms_tile_kernel(x_ref, w_ref, o_ref, *, eps: float, inv_dim: float):
    xf = x_ref[...].astype(jnp.float32)
    ms = jnp.sum(xf * xf, axis=-1, keepdims=True) * inv_dim
    inv = jax.lax.rsqrt(ms + eps)
    o_ref[...] = ((xf * inv) * w_ref[...].astype(jnp.float32)).astype(o_ref.dtype)


def kernel(x, weight):
    eps = 1e-6
    orig_shape = x.shape
    dim = orig_shape[-1]
    x2d = x.reshape(-1, dim)
    rows = x2d.shape[0]

    tile_rows = min(512, rows)
    grid = (pl.cdiv(rows, tile_rows),)

    row_spec = pl.BlockSpec((tile_rows, dim), lambda i: (i, 0))
    out = pl.pallas_call(
        functools.partial(_rms_tile_kernel, eps=eps, inv_dim=1.0 / dim),
        out_shape=jax.ShapeDtypeStruct((rows, dim), x.dtype),
        grid=grid,
        in_specs=[
            row_spec,
            pl.BlockSpec((1, dim), lambda i: (0, 0)),
        ],
        out_specs=row_spec,
        compiler_params=pltpu.CompilerParams(
            dimension_semantics=("parallel",),
            vmem_limit_bytes=60 * _MiB,
        ),
    )(x2d, weight.reshape(1, dim))

    return out.reshape(orig_shape)


# speedup vs baseline: 1.0340x; 1.0119x over previous
import functools

import jax
import jax.numpy as jnp
from jax.experimental import pallas as pl
from jax.experimental.pallas import tpu as pltpu

_MiB = 1024 * 1024


def _rms_tile_kernel(x_ref, w_ref, o_ref, *, eps: float, inv_dim: float):
    xf = x_ref[...].astype(jnp.float32)
    ms = jnp.sum(xf * xf, axis=-1, keepdims=True) * inv_dim
    inv = jax.lax.rsqrt(ms + eps)
    o_ref[...] = ((xf * inv) * w_ref[...].astype(jnp.float32)).astype(o_ref.dtype)


def kernel(x, weight):
    eps = 1e-6
    orig_shape = x.shape
    dim = orig_shape[-1]
    x2d = x.reshape(-1, dim)
    rows = x2d.shape[0]

    tile_rows = min(768, rows)
    grid = (pl.cdiv(rows, tile_rows),)

    row_spec = pl.BlockSpec((tile_rows, dim), lambda i: (i, 0))
    out = pl.pallas_call(
        functools.partial(_rms_tile_kernel, eps=eps, inv_dim=1.0 / dim),
        out_shape=jax.ShapeDtypeStruct((rows, dim), x.dtype),
        grid=grid,
        in_specs=[
            row_spec,
            pl.BlockSpec((1, dim), lambda i: (0, 0)),
        ],
        out_specs=row_spec,
        compiler_params=pltpu.CompilerParams(
            dimension_semantics=("parallel",),
            vmem_limit_bytes=60 * _MiB,
        ),
    )(x2d, weight.reshape(1, dim))

    return out.reshape(orig_shape)


# speedup vs baseline: 1.0351x; 1.0010x over previous
import functools

import jax
import jax.numpy as jnp
from jax.experimental import pallas as pl
from jax.experimental.pallas import tpu as pltpu

_MiB = 1024 * 1024


def _rms_tile_kernel(x_ref, w_ref, o_ref, *, eps: float, inv_dim: float):
    xf = x_ref[...].astype(jnp.float32)
    ms = jnp.sum(xf * xf, axis=-1, keepdims=True) * inv_dim
    inv = jax.lax.rsqrt(ms + eps)
    o_ref[...] = ((xf * inv) * w_ref[...].astype(jnp.float32)).astype(o_ref.dtype)


def kernel(x, weight):
    eps = 1e-6
    orig_shape = x.shape
    dim = orig_shape[-1]
    x2d = x.reshape(-1, dim)
    rows = x2d.shape[0]

    tile_rows = min(800, rows)
    grid = (pl.cdiv(rows, tile_rows),)

    row_spec = pl.BlockSpec((tile_rows, dim), lambda i: (i, 0))
    out = pl.pallas_call(
        functools.partial(_rms_tile_kernel, eps=eps, inv_dim=1.0 / dim),
        out_shape=jax.ShapeDtypeStruct((rows, dim), x.dtype),
        grid=grid,
        in_specs=[
            row_spec,
            pl.BlockSpec((1, dim), lambda i: (0, 0)),
        ],
        out_specs=row_spec,
        compiler_params=pltpu.CompilerParams(
            dimension_semantics=("parallel",),
            vmem_limit_bytes=(63 * _MiB + _MiB // 2),
        ),
    )(x2d, weight.reshape(1, dim))

    return out.reshape(orig_shape)
